# Initial kernel scaffold; baseline (speedup 1.0000x reference)
import jax
import jax.numpy as jnp
from jax.experimental import pallas as pl


def kernel(x, edge_index, batch, W1, b1, W2, b2, W3, b3, W_ih, W_hh, b_ih, b_hh, Wm1, bm1, Wm2, bm2):
    raise NotImplementedError("write your pallas kernel here")


# speedup vs baseline: 21.1924x; 21.1924x over previous
import functools

import jax
import jax.numpy as jnp
from jax import lax
from jax.experimental import pallas as pl
from jax.experimental.pallas import tpu as pltpu
from jax.experimental.pallas import tpu_sc as plsc

N = 10000
E = 320000
D = 128
B = 64
OUT = 10
STEPS = 4

NC, NS = 2, 16
NW = NC * NS
EPW = E // NW
KB = 80
NB = EPW // KB
NCH = 5
CH = NB // NCH
ROWB = 80
NRB = N // ROWB
RT = 1000

def _zero_vmem_2d(ref, rows, cols):
    per_row = cols // 16

    def body(k, _):
        i = k // per_row
        j = k % per_row
        ref[i, pl.ds(j * 16, 16)] = jnp.zeros((16,), jnp.float32)
        return 0

    lax.fori_loop(0, rows * per_row, body, 0)


def _blocks_roundrobin(s, total_blocks, fn):
    per = (total_blocks + NS - 1) // NS
    for j in range(per):
        blk = j * NS + s

        @pl.when(blk < total_blocks)
        def _():
            fn(blk)


@functools.cache
def _sc_kernels():
    mesh = plsc.VectorSubcoreMesh(
        core_axis_name="c", subcore_axis_name="s",
        num_cores=NC, num_subcores=NS)
    deg = functools.partial(
        pl.kernel,
        out_type=jax.ShapeDtypeStruct((NC, N, 16), jnp.float32),
        mesh=mesh,
        scratch_types=[
            pltpu.VMEM((NB, KB), jnp.int32),
            pltpu.VMEM((KB, 16), jnp.float32),
            pltpu.VMEM((ROWB, 16), jnp.float32),
            pltpu.VMEM_SHARED((N, 16), jnp.float32),
        ],
    )(_deg_body)
    scat = functools.partial(
        pl.kernel,
        out_type=jax.ShapeDtypeStruct((NC, N, D), jnp.float32),
        mesh=mesh,
        scratch_types=[
            pltpu.VMEM((CH, KB), jnp.int32),
            pltpu.VMEM((CH, KB), jnp.int32),
            pltpu.VMEM((2, KB, D), jnp.float32),
            pltpu.VMEM_SHARED((N, D), jnp.float32),
            pltpu.SemaphoreType.DMA,
        ],
    )(_scatter_body)
    return deg, scat


def _deg_kernel(dst):
    return _sc_kernels()[0](dst)


def _scatter_kernel(h, src, dst):
    return _sc_kernels()[1](h, src, dst)


def _deg_body(dst_hbm, out_hbm, didx_v, ones_v, stage_v, acc_sh):
    c = lax.axis_index("c")
    s = lax.axis_index("s")
    wid = c * NS + s

    _zero_vmem_2d(stage_v, ROWB, 16)

    def put_zero(blk):
        pltpu.sync_copy(stage_v, acc_sh.at[pl.ds(blk * ROWB, ROWB)])

    _blocks_roundrobin(s, NRB, put_zero)

    def fill_ones(i, _):
        ones_v[i, :] = jnp.full((16,), 1.0, jnp.float32)
        return 0

    lax.fori_loop(0, KB, fill_ones, 0)
    pltpu.sync_copy(dst_hbm.at[wid], didx_v)
    plsc.subcore_barrier()

    def body(i, _):
        pltpu.sync_copy(ones_v, acc_sh.at[didx_v.at[i]], add=True)
        return 0

    lax.fori_loop(0, NB, body, 0)
    plsc.subcore_barrier()

    def put_out(blk):
        sl = pl.ds(blk * ROWB, ROWB)
        pltpu.sync_copy(acc_sh.at[sl], stage_v)
        pltpu.sync_copy(stage_v, out_hbm.at[c].at[sl])

    _blocks_roundrobin(s, NRB, put_out)


def _scatter_body(h_hbm, src_hbm, dst_hbm, out_hbm,
                  sidx_v, didx_v, rows_v, acc_sh, gsem):
    c = lax.axis_index("c")
    s = lax.axis_index("s")
    wid = c * NS + s

    _zero_vmem_2d(rows_v.at[0], ROWB, D)

    def put_zero(blk):
        pltpu.sync_copy(rows_v.at[0], acc_sh.at[pl.ds(blk * ROWB, ROWB)])

    _blocks_roundrobin(s, NRB, put_zero)
    plsc.subcore_barrier()

    def chunk(ch, _):
        pltpu.sync_copy(src_hbm.at[wid, ch], sidx_v)
        pltpu.sync_copy(dst_hbm.at[wid, ch], didx_v)
        pltpu.async_copy(h_hbm.at[sidx_v.at[0]], rows_v.at[0], gsem).wait()

        def pair(k, _):
            j0 = 2 * k
            nxt = pltpu.async_copy(h_hbm.at[sidx_v.at[j0 + 1]],
                                   rows_v.at[1], gsem)
            pltpu.sync_copy(rows_v.at[0], acc_sh.at[didx_v.at[j0]], add=True)
            nxt.wait()
            nxt = pltpu.async_copy(h_hbm.at[sidx_v.at[j0 + 2]],
                                   rows_v.at[0], gsem)
            pltpu.sync_copy(rows_v.at[1],
                            acc_sh.at[didx_v.at[j0 + 1]], add=True)
            nxt.wait()
            return 0

        lax.fori_loop(0, (CH - 1) // 2, pair, 0)
        pltpu.sync_copy(rows_v.at[0], acc_sh.at[didx_v.at[CH - 1]], add=True)
        return 0

    lax.fori_loop(0, NCH, chunk, 0)
    plsc.subcore_barrier()

    def put_out(blk):
        sl = pl.ds(blk * ROWB, ROWB)
        pltpu.sync_copy(acc_sh.at[sl], rows_v.at[0])
        pltpu.sync_copy(rows_v.at[0], out_hbm.at[c].at[sl])

    _blocks_roundrobin(s, NRB, put_out)


def _dinv_body(deg_ref, out_ref):
    d = deg_ref[0, :, 0:1] + deg_ref[1, :, 0:1] + 1.0
    out_ref[...] = lax.rsqrt(d)


def _dinv(deg_parts):
    return pl.pallas_call(
        _dinv_body,
        out_shape=jax.ShapeDtypeStruct((N, 1), jnp.float32),
    )(deg_parts)


def _scale_mm_body(dinv_ref, x_ref, w_ref, out_ref):
    out_ref[...] = jnp.dot(dinv_ref[...] * x_ref[...], w_ref[...],
                           preferred_element_type=jnp.float32)


def _scale_mm(dinv, x, w):
    grid = (N // RT,)
    return pl.pallas_call(
        _scale_mm_body,
        grid=grid,
        in_specs=[
            pl.BlockSpec((RT, 1), lambda i: (i, 0)),
            pl.BlockSpec((RT, D), lambda i: (i, 0)),
            pl.BlockSpec((D, D), lambda i: (0, 0)),
        ],
        out_specs=pl.BlockSpec((RT, D), lambda i: (i, 0)),
        out_shape=jax.ShapeDtypeStruct((N, D), jnp.float32),
    )(dinv, x, w)


def _gcn_next_body(s_ref, hp_ref, dinv_ref, b_ref, w_ref, out_ref):
    t = dinv_ref[...] * (s_ref[0] + s_ref[1] + hp_ref[...]) + b_ref[...]
    t = jnp.maximum(t, 0.0)
    out_ref[...] = jnp.dot(dinv_ref[...] * t, w_ref[...],
                           preferred_element_type=jnp.float32)


def _gcn_next(s_parts, hp, dinv, b, w):
    grid = (N // RT,)
    return pl.pallas_call(
        _gcn_next_body,
        grid=grid,
        in_specs=[
            pl.BlockSpec((2, RT, D), lambda i: (0, i, 0)),
            pl.BlockSpec((RT, D), lambda i: (i, 0)),
            pl.BlockSpec((RT, 1), lambda i: (i, 0)),
            pl.BlockSpec((1, D), lambda i: (0, 0)),
            pl.BlockSpec((D, D), lambda i: (0, 0)),
        ],
        out_specs=pl.BlockSpec((RT, D), lambda i: (i, 0)),
        out_shape=jax.ShapeDtypeStruct((N, D), jnp.float32),
    )(s_parts, hp, dinv, b, w)


def _set2set_body(s_ref, hp_ref, dinv_ref, b3_ref, batch_ref,
                  wih_ref, whh_ref, bih_ref, bhh_ref,
                  wm1_ref, bm1_ref, wm2_ref, bm2_ref, out_ref):
    x = dinv_ref[...] * (s_ref[0] + s_ref[1] + hp_ref[...]) + b3_ref[...]
    x = jnp.maximum(x, 0.0)

    cols = lax.broadcasted_iota(jnp.int32, (N, B), 1)
    ohb = batch_ref[...] == cols

    wih = wih_ref[...]
    whh = whh_ref[...]
    bias = bih_ref[...] + bhh_ref[...]

    h = jnp.zeros((B, D), jnp.float32)
    cc = jnp.zeros((B, D), jnp.float32)
    q_star = jnp.zeros((B, 2 * D), jnp.float32)
    for _ in range(STEPS):
        gates = (jnp.dot(q_star, wih, preferred_element_type=jnp.float32)
                 + jnp.dot(h, whh, preferred_element_type=jnp.float32)
                 + bias)
        ig = jax.nn.sigmoid(gates[:, 0:D])
        fg = jax.nn.sigmoid(gates[:, D:2 * D])
        gg = jnp.tanh(gates[:, 2 * D:3 * D])
        og = jax.nn.sigmoid(gates[:, 3 * D:4 * D])
        cc = fg * cc + ig * gg
        h = og * jnp.tanh(cc)
        q = h
        xq = lax.dot_general(x, q, (((1,), (1,)), ((), ())),
                             preferred_element_type=jnp.float32)
        m = jnp.max(jnp.where(ohb, xq, -3e38), axis=0, keepdims=True)
        a = jnp.where(ohb, jnp.exp(xq - m), 0.0)
        ssum = jnp.sum(a, axis=0, keepdims=True)
        a = a / jnp.where(ssum > 0.0, ssum, 1.0)
        r = lax.dot_general(a, x, (((0,), (0,)), ((), ())),
                            preferred_element_type=jnp.float32)
        q_star = jnp.concatenate([q, r], axis=1)

    t = jnp.dot(q_star, wm1_ref[...], preferred_element_type=jnp.float32)
    t = jnp.maximum(t + bm1_ref[...], 0.0)
    out_ref[...] = (jnp.dot(t, wm2_ref[...], preferred_element_type=jnp.float32)
                    + bm2_ref[...])


def _set2set(s_parts, hp, dinv, b3, batch2d,
             w_ih, w_hh, b_ih, b_hh, wm1, bm1, wm2, bm2):
    return pl.pallas_call(
        _set2set_body,
        out_shape=jax.ShapeDtypeStruct((B, OUT), jnp.float32),
    )(s_parts, hp, dinv, b3, batch2d,
      w_ih, w_hh, b_ih, b_hh, wm1, bm1, wm2, bm2)


def kernel(x, edge_index, batch, W1, b1, W2, b2, W3, b3,
           W_ih, W_hh, b_ih, b_hh, Wm1, bm1, Wm2, bm2):
    src = edge_index[0].reshape(NW, NCH, CH, KB)
    dst = edge_index[1].reshape(NW, NCH, CH, KB)
    dst3 = edge_index[1].reshape(NW, NB, KB)

    deg_parts = _deg_kernel(dst3)
    dinv = _dinv(deg_parts)

    h1p = _scale_mm(dinv, x, W1)
    s1 = _scatter_kernel(h1p, src, dst)
    h2p = _gcn_next(s1, h1p, dinv, b1.reshape(1, D), W2)
    s2 = _scatter_kernel(h2p, src, dst)
    h3p = _gcn_next(s2, h2p, dinv, b2.reshape(1, D), W3)
    s3 = _scatter_kernel(h3p, src, dst)

    out = _set2set(s3, h3p, dinv, b3.reshape(1, D),
                   batch.reshape(N, 1).astype(jnp.int32),
                   W_ih, W_hh, b_ih.reshape(1, 4 * D), b_hh.reshape(1, 4 * D),
                   Wm1, bm1.reshape(1, D), Wm2, bm2.reshape(1, OUT))
    return out, jnp.zeros((), x.dtype)
